# baseline (speedup 1.0000x reference)
import functools

import jax
import jax.numpy as jnp
from jax import lax
from jax.experimental import pallas as pl
from jax.experimental.pallas import tpu as pltpu
from jax.experimental.pallas import tpu_sc as plsc

N = 10000
E = 160000
D = 256
H = 128
NC = 2
NT = 16
BK = 128
BPT = 80
BPH = BPT // 2
E_PAD = NT * BPT * BK
NRB = 2
NROW = N + 8

ROWS_A = 624
ROWS_B = N - (NT - 1) * ROWS_A

_MESH = plsc.VectorSubcoreMesh(core_axis_name="c", subcore_axis_name="s")


def _tile_slab_copy(s, src_of, dst_of):
    @pl.when(s < NT - 1)
    def _():
        pltpu.sync_copy(*_src_dst(src_of, dst_of, s * ROWS_A, ROWS_A))

    @pl.when(s == NT - 1)
    def _():
        pltpu.sync_copy(*_src_dst(src_of, dst_of, (NT - 1) * ROWS_A, ROWS_B))


def _src_dst(src_of, dst_of, r0, nr):
    return src_of(r0, nr), dst_of(r0, nr)


def _agg_half(x2, acc, sidx, didx, rows, sems):
    for k in range(NRB):
        pltpu.async_copy(x2.at[sidx.at[k]], rows[k], sems[k])

    def body(i, carry):
        for k in range(NRB):
            j = i * NRB + k
            pltpu.make_async_copy(x2.at[sidx.at[j]], rows[k],
                                  sems[k]).wait()
            pltpu.sync_copy(rows[k], acc.at[didx.at[j]], add=True)

            @pl.when(j + NRB < BPH)
            def _():
                pltpu.async_copy(x2.at[sidx.at[j + NRB]], rows[k], sems[k])
        return carry

    lax.fori_loop(0, BPH // NRB, body, 0)


def _make_agg(with_deg):
    n_out = 2 if with_deg else 1
    out_type = [jax.ShapeDtypeStruct((2 * N, H), jnp.float32)] * n_out

    def kbody(x2, src2p, dst2p, zeros, ones, *rest):
        if with_deg:
            deg_out, agg_out = rest[0], rest[1]
            refs = rest[2:]
        else:
            agg_out = rest[0]
            refs = rest[1:]
        sidx, didx, rows0, rows1, acc, sem0, sem1 = refs
        rows = (rows0, rows1)
        sems = (sem0, sem1)
        c = lax.axis_index("c")
        s = lax.axis_index("s")

        _tile_slab_copy(s, lambda r0, nr: zeros.at[pl.ds(0, nr)],
                        lambda r0, nr: acc.at[pl.ds(r0, nr)])

        if with_deg:
            pltpu.sync_copy(
                dst2p.at[pl.ds(c * (NT * BPH) + s * BPH, BPH)], didx)
            pltpu.sync_copy(ones, rows0)
            plsc.subcore_barrier()

            def dbody(i, carry):
                pltpu.sync_copy(rows0, acc.at[didx.at[i]], add=True)
                return carry

            lax.fori_loop(0, BPH, dbody, 0)
            plsc.subcore_barrier()
            _tile_slab_copy(s, lambda r0, nr: acc.at[pl.ds(r0, nr)],
                            lambda r0, nr: deg_out.at[pl.ds(c * N + r0, nr)])
            _tile_slab_copy(s, lambda r0, nr: zeros.at[pl.ds(0, nr)],
                            lambda r0, nr: acc.at[pl.ds(r0, nr)])
        plsc.subcore_barrier()

        for h in range(BPT // BPH):
            pltpu.sync_copy(
                src2p.at[pl.ds(c * (NT * BPT) + s * BPT + h * BPH, BPH)],
                sidx)
            pltpu.sync_copy(dst2p.at[pl.ds(s * BPT + h * BPH, BPH)], didx)
            _agg_half(x2, acc, sidx, didx, rows, sems)

        plsc.subcore_barrier()
        _tile_slab_copy(s, lambda r0, nr: acc.at[pl.ds(r0, nr)],
                        lambda r0, nr: agg_out.at[pl.ds(c * N + r0, nr)])

    return pl.kernel(
        kbody,
        out_type=out_type,
        mesh=_MESH,
        scratch_types=[
            pltpu.VMEM((BPH, BK), jnp.int32),
            pltpu.VMEM((BPH, BK), jnp.int32),
            pltpu.VMEM((BK, H), jnp.float32),
            pltpu.VMEM((BK, H), jnp.float32),
            pltpu.VMEM_SHARED((NROW, H), jnp.float32),
            pltpu.SemaphoreType.DMA,
            pltpu.SemaphoreType.DMA,
        ],
    )


_sc_agg_deg = _make_agg(True)
_sc_agg = _make_agg(False)


R = 1000
G = N // R

_DOT = functools.partial(
    lax.dot_general,
    dimension_numbers=(((1,), (1,)), ((), ())),
    preferred_element_type=jnp.float32,
)


def _inv_deg(dg0, dg1):
    d = dg0[:, 0:1] + dg1[:, 0:1]
    return 1.0 / jnp.maximum(d, 1.0)


def _tc1_body(agg_lo, agg_hi, dg0, dg1, x_ref, wl, wr, b, out):
    inv = _inv_deg(dg0[...], dg1[...])
    aggm = jnp.concatenate([agg_lo[...], agg_hi[...]], axis=1) * inv
    t = _DOT(aggm, wl[...]) + _DOT(x_ref[...], wr[...]) + b[...]
    h = jnp.maximum(t, 0.0)
    out[0] = h[:, :H]
    out[1] = h[:, H:]


_tc1 = pl.pallas_call(
    _tc1_body,
    grid=(G,),
    in_specs=[
        pl.BlockSpec((R, H), lambda i: (i, 0)),
        pl.BlockSpec((R, H), lambda i: (i + G, 0)),
        pl.BlockSpec((R, H), lambda i: (i, 0)),
        pl.BlockSpec((R, H), lambda i: (i + G, 0)),
        pl.BlockSpec((R, D), lambda i: (i, 0)),
        pl.BlockSpec((D, D), lambda i: (0, 0)),
        pl.BlockSpec((D, D), lambda i: (0, 0)),
        pl.BlockSpec((1, D), lambda i: (0, 0)),
    ],
    out_specs=pl.BlockSpec((2, R, H), lambda i: (0, i, 0)),
    out_shape=jax.ShapeDtypeStruct((2, N, H), jnp.float32),
)


def _tc2_body(agg_lo, agg_hi, dg0, dg1, h_lo, h_hi, wl, wr, b, wfc, bfc, out):
    inv = _inv_deg(dg0[...], dg1[...])
    aggm = jnp.concatenate([agg_lo[...], agg_hi[...]], axis=1) * inv
    hcat = jnp.concatenate([h_lo[0], h_hi[0]], axis=1)
    t = _DOT(aggm, wl[...]) + _DOT(hcat, wr[...]) + b[...]
    h2 = jnp.maximum(t, 0.0)
    out[...] = _DOT(h2, wfc[...]) + bfc[...]


_tc2 = pl.pallas_call(
    _tc2_body,
    grid=(G,),
    in_specs=[
        pl.BlockSpec((R, H), lambda i: (i, 0)),
        pl.BlockSpec((R, H), lambda i: (i + G, 0)),
        pl.BlockSpec((R, H), lambda i: (i, 0)),
        pl.BlockSpec((R, H), lambda i: (i + G, 0)),
        pl.BlockSpec((1, R, H), lambda i: (0, i, 0)),
        pl.BlockSpec((1, R, H), lambda i: (1, i, 0)),
        pl.BlockSpec((D, D), lambda i: (0, 0)),
        pl.BlockSpec((D, D), lambda i: (0, 0)),
        pl.BlockSpec((1, D), lambda i: (0, 0)),
        pl.BlockSpec((D, D), lambda i: (0, 0)),
        pl.BlockSpec((1, D), lambda i: (0, 0)),
    ],
    out_specs=pl.BlockSpec((R, D), lambda i: (i, 0)),
    out_shape=jax.ShapeDtypeStruct((N, D), jnp.float32),
)


def kernel(x, edge_index, W1l, W1r, b1, W2l, W2r, b2, Wfc, bfc):
    src = edge_index[0].astype(jnp.int32)
    dst = edge_index[1].astype(jnp.int32)
    npad = E_PAD - E
    srcp = jnp.concatenate([src, jnp.zeros((npad,), jnp.int32)])
    dstp = jnp.concatenate([dst, jnp.full((npad,), N, jnp.int32)])
    src2p = jnp.concatenate([srcp, srcp + N]).reshape(2 * NT * BPT, BK)
    dst2p = dstp.reshape(NT * BPT, BK)
    x2 = jnp.transpose(x.reshape(N, 2, H), (1, 0, 2)).reshape(2 * N, H)
    zeros = jnp.zeros((ROWS_B, H), jnp.float32)
    ones = jnp.ones((BK, H), jnp.float32)

    deg, agg1 = _sc_agg_deg(x2, src2p, dst2p, zeros, ones)
    hs = _tc1(agg1, agg1, deg, deg, x, W1l, W1r, b1.reshape(1, D))
    (agg2,) = _sc_agg(hs.reshape(2 * N, H), src2p, dst2p, zeros, ones)
    out = _tc2(agg2, agg2, deg, deg, hs, hs, W2l, W2r, b2.reshape(1, D),
               Wfc, bfc.reshape(1, D))
    return out


# speedup vs baseline: 1.7539x; 1.7539x over previous
import functools

import jax
import jax.numpy as jnp
from jax import lax
from jax.experimental import pallas as pl
from jax.experimental.pallas import tpu as pltpu
from jax.experimental.pallas import tpu_sc as plsc

N = 10000
E = 160000
D = 256
H = 128
NC = 2
NT = 16
EPT = E // NT
EPC = E // NC
EPT_DEG = EPC // NT
BK = 128
REM = EPT - (EPT // BK) * BK
REM_DEG = EPT_DEG - (EPT_DEG // BK) * BK
ROWS_A = 624
ROWS_B = N - (NT - 1) * ROWS_A

_MESH = plsc.VectorSubcoreMesh(core_axis_name="c", subcore_axis_name="s")


def _tile_slab_copy(s, src_of, dst_of):
    @pl.when(s < NT - 1)
    def _():
        pltpu.sync_copy(*src_dst(src_of, dst_of, s * ROWS_A, ROWS_A))

    @pl.when(s == NT - 1)
    def _():
        pltpu.sync_copy(*src_dst(src_of, dst_of, (NT - 1) * ROWS_A, ROWS_B))


def src_dst(src_of, dst_of, r0, nr):
    return src_of(r0, nr), dst_of(r0, nr)


@functools.partial(
    pl.kernel,
    out_type=jax.ShapeDtypeStruct((2 * N, H), jnp.float32),
    mesh=_MESH,
    scratch_types=[
        pltpu.VMEM((BK,), jnp.int32),
        pltpu.VMEM((BK,), jnp.int32),
        pltpu.VMEM((BK,), jnp.int32),
        pltpu.VMEM((BK,), jnp.int32),
        pltpu.VMEM((REM,), jnp.int32),
        pltpu.VMEM((REM,), jnp.int32),
        pltpu.VMEM((BK, H), jnp.float32),
        pltpu.VMEM((BK, H), jnp.float32),
        pltpu.VMEM((REM, H), jnp.float32),
        pltpu.VMEM_SHARED((N, H), jnp.float32),
        pltpu.SemaphoreType.DMA,
        pltpu.SemaphoreType.DMA,
    ],
)
def _sc_agg(x2, src2, dst, zeros, out, sidx_a, didx_a, sidx_b, didx_b,
            sidx_r, didx_r, rows_a, rows_b, rows_r, acc, sem_a, sem_b):
    c = lax.axis_index("c")
    s = lax.axis_index("s")
    _tile_slab_copy(s, lambda r0, nr: zeros.at[pl.ds(0, nr)],
                    lambda r0, nr: acc.at[pl.ds(r0, nr)])
    plsc.subcore_barrier()
    eb = c * E + s * EPT
    db = s * EPT
    nb = EPT // BK
    nb2 = nb // 2

    pltpu.sync_copy(src2.at[pl.ds(eb, BK)], sidx_a)
    pltpu.sync_copy(dst.at[pl.ds(db, BK)], didx_a)
    pltpu.async_copy(x2.at[sidx_a], rows_a, sem_a)

    def body(i, carry):
        offb = (2 * i + 1) * BK
        pltpu.sync_copy(src2.at[pl.ds(eb + offb, BK)], sidx_b)
        pltpu.sync_copy(dst.at[pl.ds(db + offb, BK)], didx_b)
        pltpu.async_copy(x2.at[sidx_b], rows_b, sem_b)
        pltpu.make_async_copy(x2.at[sidx_a], rows_a, sem_a).wait()
        pltpu.sync_copy(rows_a, acc.at[didx_a], add=True)

        @pl.when(i < nb2 - 1)
        def _():
            offa = (2 * i + 2) * BK
            pltpu.sync_copy(src2.at[pl.ds(eb + offa, BK)], sidx_a)
            pltpu.sync_copy(dst.at[pl.ds(db + offa, BK)], didx_a)
            pltpu.async_copy(x2.at[sidx_a], rows_a, sem_a)
        pltpu.make_async_copy(x2.at[sidx_b], rows_b, sem_b).wait()
        pltpu.sync_copy(rows_b, acc.at[didx_b], add=True)
        return carry

    lax.fori_loop(0, nb2, body, 0)
    pltpu.sync_copy(src2.at[pl.ds(eb + nb * BK, REM)], sidx_r)
    pltpu.sync_copy(dst.at[pl.ds(db + nb * BK, REM)], didx_r)
    pltpu.async_copy(x2.at[sidx_r], rows_r, sem_a).wait()
    pltpu.sync_copy(rows_r, acc.at[didx_r], add=True)
    plsc.subcore_barrier()
    _tile_slab_copy(s, lambda r0, nr: acc.at[pl.ds(r0, nr)],
                    lambda r0, nr: out.at[pl.ds(c * N + r0, nr)])


@functools.partial(
    pl.kernel,
    out_type=[jax.ShapeDtypeStruct((2 * N, H), jnp.float32),
              jax.ShapeDtypeStruct((2 * N, H), jnp.float32)],
    mesh=_MESH,
    scratch_types=[
        pltpu.VMEM((BK,), jnp.int32),
        pltpu.VMEM((BK,), jnp.int32),
        pltpu.VMEM((BK,), jnp.int32),
        pltpu.VMEM((BK,), jnp.int32),
        pltpu.VMEM((REM,), jnp.int32),
        pltpu.VMEM((REM,), jnp.int32),
        pltpu.VMEM((REM_DEG,), jnp.int32),
        pltpu.VMEM((BK, H), jnp.float32),
        pltpu.VMEM((BK, H), jnp.float32),
        pltpu.VMEM((REM, H), jnp.float32),
        pltpu.VMEM_SHARED((N, H), jnp.float32),
        pltpu.SemaphoreType.DMA,
        pltpu.SemaphoreType.DMA,
    ],
)
def _sc_agg_deg(x2, src2, dst, zeros, ones, deg_out, out, sidx_a, didx_a,
                sidx_b, didx_b, sidx_r, didx_r, didx_r8, rows_a, rows_b,
                rows_r, acc, sem_a, sem_b):
    c = lax.axis_index("c")
    s = lax.axis_index("s")
    _tile_slab_copy(s, lambda r0, nr: zeros.at[pl.ds(0, nr)],
                    lambda r0, nr: acc.at[pl.ds(r0, nr)])
    pltpu.sync_copy(ones, rows_a)
    plsc.subcore_barrier()
    dbase = c * EPC + s * EPT_DEG
    nbd = EPT_DEG // BK

    def dbody(i, carry):
        pltpu.sync_copy(dst.at[pl.ds(dbase + i * BK, BK)], didx_a)
        pltpu.sync_copy(rows_a, acc.at[didx_a], add=True)
        return carry

    lax.fori_loop(0, nbd, dbody, 0)
    pltpu.sync_copy(dst.at[pl.ds(dbase + nbd * BK, REM_DEG)], didx_r8)
    pltpu.sync_copy(rows_a.at[pl.ds(0, REM_DEG)], acc.at[didx_r8], add=True)
    plsc.subcore_barrier()
    _tile_slab_copy(s, lambda r0, nr: acc.at[pl.ds(r0, nr)],
                    lambda r0, nr: deg_out.at[pl.ds(c * N + r0, nr)])
    _tile_slab_copy(s, lambda r0, nr: zeros.at[pl.ds(0, nr)],
                    lambda r0, nr: acc.at[pl.ds(r0, nr)])
    plsc.subcore_barrier()

    eb = c * E + s * EPT
    db = s * EPT
    nb = EPT // BK
    nb2 = nb // 2

    pltpu.sync_copy(src2.at[pl.ds(eb, BK)], sidx_a)
    pltpu.sync_copy(dst.at[pl.ds(db, BK)], didx_a)
    pltpu.async_copy(x2.at[sidx_a], rows_a, sem_a)

    def body(i, carry):
        offb = (2 * i + 1) * BK
        pltpu.sync_copy(src2.at[pl.ds(eb + offb, BK)], sidx_b)
        pltpu.sync_copy(dst.at[pl.ds(db + offb, BK)], didx_b)
        pltpu.async_copy(x2.at[sidx_b], rows_b, sem_b)
        pltpu.make_async_copy(x2.at[sidx_a], rows_a, sem_a).wait()
        pltpu.sync_copy(rows_a, acc.at[didx_a], add=True)

        @pl.when(i < nb2 - 1)
        def _():
            offa = (2 * i + 2) * BK
            pltpu.sync_copy(src2.at[pl.ds(eb + offa, BK)], sidx_a)
            pltpu.sync_copy(dst.at[pl.ds(db + offa, BK)], didx_a)
            pltpu.async_copy(x2.at[sidx_a], rows_a, sem_a)
        pltpu.make_async_copy(x2.at[sidx_b], rows_b, sem_b).wait()
        pltpu.sync_copy(rows_b, acc.at[didx_b], add=True)
        return carry

    lax.fori_loop(0, nb2, body, 0)
    pltpu.sync_copy(src2.at[pl.ds(eb + nb * BK, REM)], sidx_r)
    pltpu.sync_copy(dst.at[pl.ds(db + nb * BK, REM)], didx_r)
    pltpu.async_copy(x2.at[sidx_r], rows_r, sem_a).wait()
    pltpu.sync_copy(rows_r, acc.at[didx_r], add=True)
    plsc.subcore_barrier()
    _tile_slab_copy(s, lambda r0, nr: acc.at[pl.ds(r0, nr)],
                    lambda r0, nr: out.at[pl.ds(c * N + r0, nr)])


R = 1000
G = N // R

_DOT = functools.partial(
    lax.dot_general,
    dimension_numbers=(((1,), (1,)), ((), ())),
    preferred_element_type=jnp.float32,
)


def _inv_deg(dg0, dg1):
    d = dg0[:, 0:1] + dg1[:, 0:1]
    return 1.0 / jnp.maximum(d, 1.0)


def _tc1_body(agg_lo, agg_hi, dg0, dg1, x_ref, wl, wr, b, out):
    inv = _inv_deg(dg0[...], dg1[...])
    aggm = jnp.concatenate([agg_lo[...], agg_hi[...]], axis=1) * inv
    t = _DOT(aggm, wl[...]) + _DOT(x_ref[...], wr[...]) + b[...]
    h = jnp.maximum(t, 0.0)
    out[0] = h[:, :H]
    out[1] = h[:, H:]


_tc1 = pl.pallas_call(
    _tc1_body,
    grid=(G,),
    in_specs=[
        pl.BlockSpec((R, H), lambda i: (i, 0)),
        pl.BlockSpec((R, H), lambda i: (i + G, 0)),
        pl.BlockSpec((R, H), lambda i: (i, 0)),
        pl.BlockSpec((R, H), lambda i: (i + G, 0)),
        pl.BlockSpec((R, D), lambda i: (i, 0)),
        pl.BlockSpec((D, D), lambda i: (0, 0)),
        pl.BlockSpec((D, D), lambda i: (0, 0)),
        pl.BlockSpec((1, D), lambda i: (0, 0)),
    ],
    out_specs=pl.BlockSpec((2, R, H), lambda i: (0, i, 0)),
    out_shape=jax.ShapeDtypeStruct((2, N, H), jnp.float32),
)


def _tc2_body(agg_lo, agg_hi, dg0, dg1, h_lo, h_hi, wl, wr, b, wfc, bfc, out):
    inv = _inv_deg(dg0[...], dg1[...])
    aggm = jnp.concatenate([agg_lo[...], agg_hi[...]], axis=1) * inv
    hcat = jnp.concatenate([h_lo[0], h_hi[0]], axis=1)
    t = _DOT(aggm, wl[...]) + _DOT(hcat, wr[...]) + b[...]
    h2 = jnp.maximum(t, 0.0)
    out[...] = _DOT(h2, wfc[...]) + bfc[...]


_tc2 = pl.pallas_call(
    _tc2_body,
    grid=(G,),
    in_specs=[
        pl.BlockSpec((R, H), lambda i: (i, 0)),
        pl.BlockSpec((R, H), lambda i: (i + G, 0)),
        pl.BlockSpec((R, H), lambda i: (i, 0)),
        pl.BlockSpec((R, H), lambda i: (i + G, 0)),
        pl.BlockSpec((1, R, H), lambda i: (0, i, 0)),
        pl.BlockSpec((1, R, H), lambda i: (1, i, 0)),
        pl.BlockSpec((D, D), lambda i: (0, 0)),
        pl.BlockSpec((D, D), lambda i: (0, 0)),
        pl.BlockSpec((1, D), lambda i: (0, 0)),
        pl.BlockSpec((D, D), lambda i: (0, 0)),
        pl.BlockSpec((1, D), lambda i: (0, 0)),
    ],
    out_specs=pl.BlockSpec((R, D), lambda i: (i, 0)),
    out_shape=jax.ShapeDtypeStruct((N, D), jnp.float32),
)


def kernel(x, edge_index, W1l, W1r, b1, W2l, W2r, b2, Wfc, bfc):
    src = edge_index[0].astype(jnp.int32)
    dst = edge_index[1].astype(jnp.int32)
    src2 = jnp.concatenate([src, src + N])
    x2 = jnp.transpose(x.reshape(N, 2, H), (1, 0, 2)).reshape(2 * N, H)
    zeros = jnp.zeros((ROWS_B, H), jnp.float32)
    ones = jnp.ones((BK, H), jnp.float32)

    deg, agg1 = _sc_agg_deg(x2, src2, dst, zeros, ones)
    hs = _tc1(agg1, agg1, deg, deg, x, W1l, W1r, b1.reshape(1, D))
    agg2 = _sc_agg(hs.reshape(2 * N, H), src2, dst, zeros)
    out = _tc2(agg2, agg2, deg, deg, hs, hs, W2l, W2r, b2.reshape(1, D),
               Wfc, bfc.reshape(1, D))
    return out
